# baseline (speedup 1.0000x reference)
import jax
import jax.numpy as jnp
from jax.experimental import pallas as pl
from jax.experimental.pallas import tpu as pltpu

R = 4
N = 4096
DIN = 256
DOUT = 256
BM = 512
NBUF = 4
MT = N // BM
T = R * MT


def _rgcn_body(x_ref, w_ref, b_ref, adj_hbm, out_ref, sup_ref, abuf, sem):
    def start_dma(t, slot):
        r = t // MT
        m = t % MT
        pltpu.make_async_copy(
            adj_hbm.at[r, pl.ds(m * BM, BM), :],
            abuf.at[slot],
            sem.at[slot],
        ).start()

    for t in range(NBUF):
        start_dma(t, t)

    for r in range(R):
        s = jax.lax.dot_general(
            x_ref[...], w_ref[r], (((1,), (1,)), ((), ())),
            preferred_element_type=jnp.float32)
        sup_ref[r] = s + b_ref[r]

    def body(t, carry):
        slot = jax.lax.rem(t, NBUF)
        r = t // MT
        m = t % MT
        pltpu.make_async_copy(
            adj_hbm.at[0, pl.ds(0, BM), :], abuf.at[slot], sem.at[slot]
        ).wait()
        contrib = jnp.dot(abuf[slot], sup_ref[r],
                          preferred_element_type=jnp.float32)

        @pl.when(t + NBUF < T)
        def _():
            start_dma(t + NBUF, slot)

        rows = pl.ds(m * BM, BM)

        @pl.when(r == 0)
        def _():
            out_ref[rows, :] = contrib

        @pl.when(jnp.logical_and(r > 0, r < R - 1))
        def _():
            out_ref[rows, :] = out_ref[rows, :] + contrib

        @pl.when(r == R - 1)
        def _():
            out_ref[rows, :] = jnp.tanh(out_ref[rows, :] + contrib)

        return carry

    jax.lax.fori_loop(0, T, body, 0)


@jax.jit
def kernel(input, adjs, W, b):
    b3 = b.reshape(R, 1, DOUT)
    return pl.pallas_call(
        _rgcn_body,
        in_specs=[
            pl.BlockSpec((N, DIN), lambda: (0, 0)),
            pl.BlockSpec((R, DOUT, DIN), lambda: (0, 0, 0)),
            pl.BlockSpec((R, 1, DOUT), lambda: (0, 0, 0)),
            pl.BlockSpec(memory_space=pl.ANY),
        ],
        out_specs=pl.BlockSpec((N, DOUT), lambda: (0, 0)),
        out_shape=jax.ShapeDtypeStruct((N, DOUT), jnp.float32),
        scratch_shapes=[
            pltpu.VMEM((R, N, DOUT), jnp.float32),
            pltpu.VMEM((NBUF, BM, N), jnp.float32),
            pltpu.SemaphoreType.DMA((NBUF,)),
        ],
        compiler_params=pltpu.CompilerParams(
            vmem_limit_bytes=100 * 1024 * 1024,
        ),
    )(input, W, b3, adjs)


# speedup vs baseline: 1.0150x; 1.0150x over previous
import jax
import jax.numpy as jnp
from jax.experimental import pallas as pl
from jax.experimental.pallas import tpu as pltpu

R = 4
N = 4096
DIN = 256
DOUT = 256
BM = 128
NBUF = 8
MT = N // BM
T = R * MT


def _rgcn_body(x_ref, w_ref, b_ref, adj_hbm, out_ref, sup_ref, abuf, sem):
    def start_dma(t, slot):
        r = t // MT
        m = t % MT
        pltpu.make_async_copy(
            adj_hbm.at[r, pl.ds(m * BM, BM), :],
            abuf.at[slot],
            sem.at[slot],
        ).start()

    for t in range(NBUF):
        start_dma(t, t)

    for r in range(R):
        s = jax.lax.dot_general(
            x_ref[...], w_ref[r], (((1,), (1,)), ((), ())),
            preferred_element_type=jnp.float32)
        sup_ref[r] = s + b_ref[r]

    def body(t, carry):
        slot = jax.lax.rem(t, NBUF)
        r = t // MT
        m = t % MT
        pltpu.make_async_copy(
            adj_hbm.at[0, pl.ds(0, BM), :], abuf.at[slot], sem.at[slot]
        ).wait()
        contrib = jnp.dot(abuf[slot], sup_ref[r],
                          preferred_element_type=jnp.float32)

        @pl.when(t + NBUF < T)
        def _():
            start_dma(t + NBUF, slot)

        rows = pl.ds(m * BM, BM)

        @pl.when(r == 0)
        def _():
            out_ref[rows, :] = contrib

        @pl.when(jnp.logical_and(r > 0, r < R - 1))
        def _():
            out_ref[rows, :] = out_ref[rows, :] + contrib

        @pl.when(r == R - 1)
        def _():
            out_ref[rows, :] = jnp.tanh(out_ref[rows, :] + contrib)

        return carry

    jax.lax.fori_loop(0, T, body, 0)


@jax.jit
def kernel(input, adjs, W, b):
    b3 = b.reshape(R, 1, DOUT)
    return pl.pallas_call(
        _rgcn_body,
        in_specs=[
            pl.BlockSpec((N, DIN), lambda: (0, 0)),
            pl.BlockSpec((R, DOUT, DIN), lambda: (0, 0, 0)),
            pl.BlockSpec((R, 1, DOUT), lambda: (0, 0, 0)),
            pl.BlockSpec(memory_space=pl.ANY),
        ],
        out_specs=pl.BlockSpec((N, DOUT), lambda: (0, 0)),
        out_shape=jax.ShapeDtypeStruct((N, DOUT), jnp.float32),
        scratch_shapes=[
            pltpu.VMEM((R, N, DOUT), jnp.float32),
            pltpu.VMEM((NBUF, BM, N), jnp.float32),
            pltpu.SemaphoreType.DMA((NBUF,)),
        ],
        compiler_params=pltpu.CompilerParams(
            vmem_limit_bytes=100 * 1024 * 1024,
        ),
    )(input, W, b3, adjs)


# speedup vs baseline: 1.0541x; 1.0385x over previous
import jax
import jax.numpy as jnp
from jax.experimental import pallas as pl
from jax.experimental.pallas import tpu as pltpu

R = 4
N = 4096
DIN = 256
DOUT = 256
BM = 256
NBUF = 4
MT = N // BM
T = R * MT


def _rgcn_body(x_ref, w_ref, b_ref, adj_hbm, out_ref, sup_ref, abuf, sem):
    def start_dma(t, slot):
        r = t // MT
        m = t % MT
        pltpu.make_async_copy(
            adj_hbm.at[r, pl.ds(m * BM, BM), :],
            abuf.at[slot],
            sem.at[slot],
        ).start()

    for t in range(NBUF):
        start_dma(t, t)

    for r in range(R):
        s = jax.lax.dot_general(
            x_ref[...], w_ref[r], (((1,), (1,)), ((), ())),
            preferred_element_type=jnp.float32)
        sup_ref[r] = s + b_ref[r]

    def body(t, carry):
        slot = jax.lax.rem(t, NBUF)
        r = t // MT
        m = t % MT
        pltpu.make_async_copy(
            adj_hbm.at[0, pl.ds(0, BM), :], abuf.at[slot], sem.at[slot]
        ).wait()
        contrib = jnp.dot(abuf[slot], sup_ref[r],
                          preferred_element_type=jnp.float32)

        @pl.when(t + NBUF < T)
        def _():
            start_dma(t + NBUF, slot)

        rows = pl.ds(m * BM, BM)

        @pl.when(r == 0)
        def _():
            out_ref[rows, :] = contrib

        @pl.when(jnp.logical_and(r > 0, r < R - 1))
        def _():
            out_ref[rows, :] = out_ref[rows, :] + contrib

        @pl.when(r == R - 1)
        def _():
            out_ref[rows, :] = jnp.tanh(out_ref[rows, :] + contrib)

        return carry

    jax.lax.fori_loop(0, T, body, 0)


@jax.jit
def kernel(input, adjs, W, b):
    b3 = b.reshape(R, 1, DOUT)
    return pl.pallas_call(
        _rgcn_body,
        in_specs=[
            pl.BlockSpec((N, DIN), lambda: (0, 0)),
            pl.BlockSpec((R, DOUT, DIN), lambda: (0, 0, 0)),
            pl.BlockSpec((R, 1, DOUT), lambda: (0, 0, 0)),
            pl.BlockSpec(memory_space=pl.ANY),
        ],
        out_specs=pl.BlockSpec((N, DOUT), lambda: (0, 0)),
        out_shape=jax.ShapeDtypeStruct((N, DOUT), jnp.float32),
        scratch_shapes=[
            pltpu.VMEM((R, N, DOUT), jnp.float32),
            pltpu.VMEM((NBUF, BM, N), jnp.float32),
            pltpu.SemaphoreType.DMA((NBUF,)),
        ],
        compiler_params=pltpu.CompilerParams(
            vmem_limit_bytes=100 * 1024 * 1024,
        ),
    )(input, W, b3, adjs)
